# baseline (speedup 1.0000x reference)
import functools
import math

import jax
import jax.numpy as jnp
from jax import lax
from jax.experimental import pallas as pl
from jax.experimental.pallas import tpu as pltpu
from jax.experimental.pallas import tpu_sc as plsc

S = 2048
D = 768
E = 16
BT = 256
NUM_SLOTS = S + E * BT
NB = NUM_SLOTS // BT
CH = 128


_XC = 4
_XCS = S // _XC


def _router_body(x_hbm, wg_ref, bg_ref, probs_ref, slot_ref, meta_ref,
                 xq, oh_scr, rank_scr, sq0, sq1, sq2, sq3):
    sqs = (sq0, sq1, sq2, sq3)

    def _fetch(c):
        return pltpu.make_async_copy(x_hbm.at[pl.ds(c * _XCS, _XCS)],
                                     xq.at[c], sqs[c])

    for c in range(_XC):
        _fetch(c).start()

    lane = lax.broadcasted_iota(jnp.int32, (_XCS, E), 1).astype(jnp.float32)
    r = lax.broadcasted_iota(jnp.int32, (_XCS, _XCS), 0)
    c2 = lax.broadcasted_iota(jnp.int32, (_XCS, _XCS), 1)
    tril = (c2 <= r).astype(jnp.float32)
    run = jnp.zeros((1, E), jnp.float32)
    for c in range(_XC):
        _fetch(c).wait()
        x = xq[c]
        logits = lax.dot_general(x, wg_ref[...],
                                 (((1,), (1,)), ((), ())),
                                 preferred_element_type=jnp.float32)
        logits = logits + bg_ref[...]
        m = jnp.max(logits, axis=1, keepdims=True)
        ex = jnp.exp(logits - m)
        probs = ex / jnp.sum(ex, axis=1, keepdims=True)
        probs_ref[pl.ds(c * _XCS, _XCS), :] = probs
        pm = jnp.max(probs, axis=1, keepdims=True)
        eidx = jnp.min(jnp.where(probs == pm, lane, float(E)), axis=1,
                       keepdims=True)
        onehot = (lane == eidx).astype(jnp.float32)
        oh_scr[pl.ds(c * _XCS, _XCS), :] = onehot
        prefix = jnp.dot(tril, onehot,
                         preferred_element_type=jnp.float32) + run
        rank_scr[pl.ds(c * _XCS, _XCS), :] = (
            jnp.sum(onehot * (prefix - 1.0), axis=1, keepdims=True))
        run = run + jnp.sum(onehot, axis=0, keepdims=True)

    onehot = oh_scr[...]
    rank = rank_scr[...]
    counts = run.astype(jnp.int32)
    _SH = BT.bit_length() - 1
    padded = ((counts + (BT - 1)) >> _SH) << _SH
    r16 = lax.broadcasted_iota(jnp.int32, (E, E), 0)
    c16 = lax.broadcasted_iota(jnp.int32, (E, E), 1)
    supper = (r16 < c16).astype(jnp.float32)
    pstarts = jnp.dot(padded.astype(jnp.float32), supper,
                      preferred_element_type=jnp.float32)

    slot = jnp.sum(onehot * pstarts, axis=1, keepdims=True) + rank
    slot_ref[...] = slot.astype(jnp.int32).reshape(S)

    bstart = lax.broadcasted_iota(jnp.int32, (NB, 1), 0).astype(jnp.float32) * float(BT)
    owner = jnp.sum((pstarts <= bstart).astype(jnp.float32), axis=1,
                    keepdims=True) - 1.0

    prev = jnp.concatenate([jnp.full((1, 1), -1.0), owner[:-1]], axis=0)
    loadf = (owner != prev).astype(jnp.float32)
    rb = lax.broadcasted_iota(jnp.int32, (NB, NB), 0)
    cb = lax.broadcasted_iota(jnp.int32, (NB, NB), 1)
    trilb = (cb <= rb).astype(jnp.float32)
    seg = jnp.dot(trilb, loadf, preferred_element_type=jnp.float32) - 1.0
    third = jnp.floor(seg * (1.0 / 3.0) + 1e-6)
    parity = seg - 3.0 * third
    jmat = cb.astype(jnp.float32)
    ident = (rb == cb).astype(jnp.float32)
    loadrow = lax.dot_general(loadf, ident, (((0,), (0,)), ((), ())),
                              preferred_element_type=jnp.float32)
    cand = jnp.where((cb > rb) & (loadrow > 0.5), jmat, float(NB))
    nxt = jnp.min(cand, axis=1, keepdims=True)
    has_nxt = (nxt < float(NB)).astype(jnp.float32)
    sel = (jmat == nxt).astype(jnp.float32)
    nxt_e = jnp.dot(sel, owner, preferred_element_type=jnp.float32)
    nxt2 = jnp.dot(sel, nxt, preferred_element_type=jnp.float32)
    has2 = jnp.dot(sel, has_nxt, preferred_element_type=jnp.float32)
    sel2 = (jmat == nxt2).astype(jnp.float32)
    nxt2_e = jnp.dot(sel2, owner, preferred_element_type=jnp.float32)
    total = jnp.sum(padded.astype(jnp.float32), axis=1, keepdims=True)
    active = (bstart < total).astype(jnp.float32)
    meta = jnp.concatenate([owner, loadf, parity, nxt_e, has_nxt,
                            nxt2_e, has2, active], axis=1)
    meta_ref[...] = meta.astype(jnp.int32)


def _router(x2, Wg, bg2):
    return pl.pallas_call(
        _router_body,
        in_specs=[
            pl.BlockSpec(memory_space=pl.ANY),
            pl.BlockSpec((E, D), lambda: (0, 0)),
            pl.BlockSpec((1, E), lambda: (0, 0)),
        ],
        out_shape=[
            jax.ShapeDtypeStruct((S, E), jnp.float32),
            jax.ShapeDtypeStruct((S,), jnp.int32),
            jax.ShapeDtypeStruct((NB, 8), jnp.int32),
        ],
        scratch_shapes=[
            pltpu.VMEM((_XC, _XCS, D), jnp.float32),
            pltpu.VMEM((S, E), jnp.float32),
            pltpu.VMEM((S, 1), jnp.float32),
            pltpu.SemaphoreType.DMA,
            pltpu.SemaphoreType.DMA,
            pltpu.SemaphoreType.DMA,
            pltpu.SemaphoreType.DMA,
        ],
    )(x2, Wg, bg2)


def _ffn_body(meta_ref, xs_hbm, w1_hbm, b1_hbm, w2_hbm, b2_hbm, ys_hbm,
              w1_scr, w2_scr, sem_a, sem_b, sem_c,
              x_scr, y_scr, b1_scr, b2_scr,
              sx0, sx1, sx2, sy0, sy1, sy2, sbias):
    b = pl.program_id(0)
    cur_e = meta_ref[b, 0]
    ld = meta_ref[b, 1]
    p = meta_ref[b, 2]
    nxt_e = meta_ref[b, 3]
    has_nxt = meta_ref[b, 4]
    nxt2_e = meta_ref[b, 5]
    has2 = meta_ref[b, 6]
    sems = (sem_a, sem_b, sem_c)

    HD = D // 2

    def _halves(e, slot_i):
        for w_hbm, w_scr in ((w1_hbm, w1_scr), (w2_hbm, w2_scr)):
            for h0 in (0, HD):
                yield pltpu.make_async_copy(
                    w_hbm.at[e, pl.ds(h0, HD)],
                    w_scr.at[slot_i, pl.ds(h0, HD)],
                    sems[slot_i])

    def _start(e, slot_i):
        for cp in _halves(e, slot_i):
            cp.start()

    def _wait(e, slot_i):
        for cp in _halves(e, slot_i):
            cp.wait()

    sxs = (sx0, sx1, sx2)
    sys_ = (sy0, sy1, sy2)

    def _xfetch(j, buf_i):
        return pltpu.make_async_copy(xs_hbm.at[pl.ds(j * BT, BT)],
                                     x_scr.at[buf_i], sxs[buf_i])

    def _ywrite(j, buf_i):
        return pltpu.make_async_copy(y_scr.at[buf_i],
                                     ys_hbm.at[pl.ds(j * BT, BT)],
                                     sys_[buf_i])

    @pl.when(b == 0)
    def _():
        _xfetch(0, 0).start()
        _xfetch(1, 1).start()
        pltpu.make_async_copy(b1_hbm, b1_scr, sbias).start()
        pltpu.make_async_copy(b2_hbm, b2_scr, sbias).start()
        pltpu.make_async_copy(b1_hbm, b1_scr, sbias).wait()
        pltpu.make_async_copy(b2_hbm, b2_scr, sbias).wait()

    @pl.when((b < NB - 2))
    def _():
        for i in range(3):
            @pl.when((lax.rem(b + 2, 3) == i) & (meta_ref[b + 2, 7] == 1))
            def _(i=i):
                _xfetch(b + 2, i).start()

    @pl.when(b == 0)
    def _():
        _start(cur_e, 0)

    @pl.when((b == 0) & (has_nxt == 1))
    def _():
        _start(nxt_e, 1)

    for pp in range(3):
        @pl.when((ld == 1) & (has2 == 1) & (p == pp))
        def _(pp=pp):
            _start(nxt2_e, (pp + 2) % 3)

    for pp in range(3):
        @pl.when((ld == 1) & (p == pp))
        def _(pp=pp):
            _wait(cur_e, pp)

    p3 = lax.rem(b, 3)
    act = meta_ref[b, 7]
    for i in range(3):
        @pl.when((p3 == i) & (act == 1))
        def _(i=i):
            _xfetch(b, i).wait()

    @pl.when(meta_ref[b, 7] == 1)
    def _():
        for i in range(3):
            @pl.when(p3 == i)
            def _(i=i):
                x = x_scr[i]
                h = jnp.dot(x, w1_scr[p],
                            preferred_element_type=jnp.float32)
                h = h + b1_scr[cur_e]
                h = 0.5 * h * (1.0 + lax.erf(h * (1.0 / math.sqrt(2.0))))
                y = jnp.dot(h, w2_scr[p],
                            preferred_element_type=jnp.float32)
                y_scr[i] = y + b2_scr[cur_e]

    for i in range(3):
        @pl.when((p3 == i) & (act == 1))
        def _(i=i):
            _ywrite(b, i).start()

    @pl.when(b >= 2)
    def _():
        for i in range(3):
            @pl.when((lax.rem(b - 2, 3) == i) & (meta_ref[b - 2, 7] == 1))
            def _(i=i):
                _ywrite(b - 2, i).wait()

    @pl.when(b == NB - 1)
    def _():
        for j in (NB - 2, NB - 1):
            @pl.when(meta_ref[j, 7] == 1)
            def _(j=j):
                _ywrite(j, j % 3).wait()


def _ffn(meta, xs, W1, b1, W2, b2):
    grid_spec = pltpu.PrefetchScalarGridSpec(
        num_scalar_prefetch=1,
        grid=(NB,),
        in_specs=[
            pl.BlockSpec(memory_space=pl.ANY),
            pl.BlockSpec(memory_space=pl.ANY),
            pl.BlockSpec(memory_space=pl.ANY),
            pl.BlockSpec(memory_space=pl.ANY),
            pl.BlockSpec(memory_space=pl.ANY),
        ],
        out_specs=pl.BlockSpec(memory_space=pl.ANY),
        scratch_shapes=[
            pltpu.VMEM((3, D, D), jnp.float32),
            pltpu.VMEM((3, D, D), jnp.float32),
            pltpu.SemaphoreType.DMA,
            pltpu.SemaphoreType.DMA,
            pltpu.SemaphoreType.DMA,
            pltpu.VMEM((3, BT, D), jnp.float32),
            pltpu.VMEM((3, BT, D), jnp.float32),
            pltpu.VMEM((E, D), jnp.float32),
            pltpu.VMEM((E, D), jnp.float32),
            pltpu.SemaphoreType.DMA,
            pltpu.SemaphoreType.DMA,
            pltpu.SemaphoreType.DMA,
            pltpu.SemaphoreType.DMA,
            pltpu.SemaphoreType.DMA,
            pltpu.SemaphoreType.DMA,
            pltpu.SemaphoreType.DMA,
        ],
    )
    return pl.pallas_call(
        _ffn_body,
        grid_spec=grid_spec,
        out_shape=jax.ShapeDtypeStruct((NUM_SLOTS, D), jnp.float32),
    )(meta, xs, W1, b1, W2, b2)


_SC_CORES = 2
_SC_SUBCORES = 16


@functools.cache
def _make_sc_kernels():
    nw = _SC_CORES * _SC_SUBCORES
    bw = S // nw
    mesh = plsc.VectorSubcoreMesh(core_axis_name="c", subcore_axis_name="s")
    scratch = [
        pltpu.VMEM((bw,), jnp.int32),
        pltpu.VMEM((bw, D), jnp.float32),
        pltpu.SemaphoreType.DMA,
    ]

    @functools.partial(
        pl.kernel, mesh=mesh,
        out_type=jax.ShapeDtypeStruct((NUM_SLOTS, D), jnp.float32),
        scratch_types=scratch,
    )
    def dispatch(x_hbm, slot_hbm, xs_hbm, idx_v, rows_v, sem):
        wid = lax.axis_index("s") * _SC_CORES + lax.axis_index("c")
        base = wid * bw
        pltpu.sync_copy(slot_hbm.at[pl.ds(base, bw)], idx_v)
        pltpu.sync_copy(x_hbm.at[pl.ds(base, bw)], rows_v)
        pltpu.async_copy(rows_v, xs_hbm.at[idx_v], sem).wait()

    @functools.partial(
        pl.kernel, mesh=mesh,
        out_type=jax.ShapeDtypeStruct((S, D), jnp.float32),
        scratch_types=scratch,
    )
    def combine(ys_hbm, slot_hbm, out_hbm, idx_v, rows_v, sem):
        wid = lax.axis_index("s") * _SC_CORES + lax.axis_index("c")
        base = wid * bw
        pltpu.sync_copy(slot_hbm.at[pl.ds(base, bw)], idx_v)
        pltpu.async_copy(ys_hbm.at[idx_v], rows_v, sem).wait()
        pltpu.sync_copy(rows_v, out_hbm.at[pl.ds(base, bw)])

    return dispatch, combine


def kernel(x, Wg, bg, W1, b1, W2, b2):
    _dispatch, _combine = _make_sc_kernels()
    x2 = x.reshape(S, D)
    probs, slot, meta = _router(x2, Wg, bg.reshape(1, E))
    xs = _dispatch(x2, slot)
    ys = _ffn(meta, xs, W1, b1, W2, b2)
    out = _combine(ys, slot)
    return out.reshape(x.shape), probs.reshape(x.shape[0], S, E)


# speedup vs baseline: 1.0401x; 1.0401x over previous
import functools
import math

import jax
import jax.numpy as jnp
from jax import lax
from jax.experimental import pallas as pl
from jax.experimental.pallas import tpu as pltpu
from jax.experimental.pallas import tpu_sc as plsc

S = 2048
D = 768
E = 16
BT = 256
NUM_SLOTS = S + E * BT
NB = NUM_SLOTS // BT
CH = 128


def _router_body(x_ref, wg_ref, bg_ref, probs_ref, slot_ref, meta_ref):
    x = x_ref[...]
    logits = lax.dot_general(x, wg_ref[...],
                             (((1,), (1,)), ((), ())),
                             preferred_element_type=jnp.float32)
    logits = logits + bg_ref[...]
    m = jnp.max(logits, axis=1, keepdims=True)
    ex = jnp.exp(logits - m)
    probs = ex / jnp.sum(ex, axis=1, keepdims=True)
    probs_ref[...] = probs

    lane = lax.broadcasted_iota(jnp.int32, (S, E), 1).astype(jnp.float32)
    pm = jnp.max(probs, axis=1, keepdims=True)
    eidx = jnp.min(jnp.where(probs == pm, lane, float(E)), axis=1,
                   keepdims=True)
    onehot = (lane == eidx).astype(jnp.float32)

    r = lax.broadcasted_iota(jnp.int32, (CH, CH), 0)
    c = lax.broadcasted_iota(jnp.int32, (CH, CH), 1)
    tril = (c <= r).astype(jnp.float32)
    run = jnp.zeros((1, E), jnp.float32)
    chunks = []
    for i in range(S // CH):
        oh = onehot[i * CH:(i + 1) * CH]
        chunks.append(jnp.dot(tril, oh, preferred_element_type=jnp.float32)
                      + run)
        run = run + jnp.sum(oh, axis=0, keepdims=True)
    prefix = jnp.concatenate(chunks, axis=0)
    rank = jnp.sum(onehot * (prefix - 1.0), axis=1, keepdims=True)

    counts = run.astype(jnp.int32)
    _SH = BT.bit_length() - 1
    padded = ((counts + (BT - 1)) >> _SH) << _SH
    r16 = lax.broadcasted_iota(jnp.int32, (E, E), 0)
    c16 = lax.broadcasted_iota(jnp.int32, (E, E), 1)
    supper = (r16 < c16).astype(jnp.float32)
    pstarts = jnp.dot(padded.astype(jnp.float32), supper,
                      preferred_element_type=jnp.float32)

    slot = jnp.sum(onehot * pstarts, axis=1, keepdims=True) + rank
    slot_ref[...] = slot.astype(jnp.int32).reshape(S)

    bstart = lax.broadcasted_iota(jnp.int32, (NB, 1), 0).astype(jnp.float32) * float(BT)
    owner = jnp.sum((pstarts <= bstart).astype(jnp.float32), axis=1,
                    keepdims=True) - 1.0

    prev = jnp.concatenate([jnp.full((1, 1), -1.0), owner[:-1]], axis=0)
    loadf = (owner != prev).astype(jnp.float32)
    rb = lax.broadcasted_iota(jnp.int32, (NB, NB), 0)
    cb = lax.broadcasted_iota(jnp.int32, (NB, NB), 1)
    trilb = (cb <= rb).astype(jnp.float32)
    seg = jnp.dot(trilb, loadf, preferred_element_type=jnp.float32) - 1.0
    third = jnp.floor(seg * (1.0 / 3.0) + 1e-6)
    parity = seg - 3.0 * third
    jmat = cb.astype(jnp.float32)
    ident = (rb == cb).astype(jnp.float32)
    loadrow = lax.dot_general(loadf, ident, (((0,), (0,)), ((), ())),
                              preferred_element_type=jnp.float32)
    cand = jnp.where((cb > rb) & (loadrow > 0.5), jmat, float(NB))
    nxt = jnp.min(cand, axis=1, keepdims=True)
    has_nxt = (nxt < float(NB)).astype(jnp.float32)
    sel = (jmat == nxt).astype(jnp.float32)
    nxt_e = jnp.dot(sel, owner, preferred_element_type=jnp.float32)
    nxt2 = jnp.dot(sel, nxt, preferred_element_type=jnp.float32)
    has2 = jnp.dot(sel, has_nxt, preferred_element_type=jnp.float32)
    sel2 = (jmat == nxt2).astype(jnp.float32)
    nxt2_e = jnp.dot(sel2, owner, preferred_element_type=jnp.float32)
    total = jnp.sum(padded.astype(jnp.float32), axis=1, keepdims=True)
    active = (bstart < total).astype(jnp.float32)
    meta = jnp.concatenate([owner, loadf, parity, nxt_e, has_nxt,
                            nxt2_e, has2, active], axis=1)
    meta_ref[...] = meta.astype(jnp.int32)


def _router(x2, Wg, bg2):
    return pl.pallas_call(
        _router_body,
        out_shape=[
            jax.ShapeDtypeStruct((S, E), jnp.float32),
            jax.ShapeDtypeStruct((S,), jnp.int32),
            jax.ShapeDtypeStruct((NB, 8), jnp.int32),
        ],
    )(x2, Wg, bg2)


def _ffn_body(meta_ref, xs_hbm, w1_hbm, b1_hbm, w2_hbm, b2_hbm, ys_hbm,
              w1_scr, w2_scr, sem_a, sem_b, sem_c,
              x_scr, y_scr, b1_scr, b2_scr,
              sx0, sx1, sx2, sy0, sy1, sy2, sbias):
    b = pl.program_id(0)
    cur_e = meta_ref[b, 0]
    ld = meta_ref[b, 1]
    p = meta_ref[b, 2]
    nxt_e = meta_ref[b, 3]
    has_nxt = meta_ref[b, 4]
    nxt2_e = meta_ref[b, 5]
    has2 = meta_ref[b, 6]
    sems = (sem_a, sem_b, sem_c)

    HD = D // 2

    def _halves(e, slot_i):
        for w_hbm, w_scr in ((w1_hbm, w1_scr), (w2_hbm, w2_scr)):
            for h0 in (0, HD):
                yield pltpu.make_async_copy(
                    w_hbm.at[e, pl.ds(h0, HD)],
                    w_scr.at[slot_i, pl.ds(h0, HD)],
                    sems[slot_i])

    def _start(e, slot_i):
        for cp in _halves(e, slot_i):
            cp.start()

    def _wait(e, slot_i):
        for cp in _halves(e, slot_i):
            cp.wait()

    sxs = (sx0, sx1, sx2)
    sys_ = (sy0, sy1, sy2)

    def _xfetch(j, buf_i):
        return pltpu.make_async_copy(xs_hbm.at[pl.ds(j * BT, BT)],
                                     x_scr.at[buf_i], sxs[buf_i])

    def _ywrite(j, buf_i):
        return pltpu.make_async_copy(y_scr.at[buf_i],
                                     ys_hbm.at[pl.ds(j * BT, BT)],
                                     sys_[buf_i])

    @pl.when(b == 0)
    def _():
        _xfetch(0, 0).start()
        _xfetch(1, 1).start()
        pltpu.make_async_copy(b1_hbm, b1_scr, sbias).start()
        pltpu.make_async_copy(b2_hbm, b2_scr, sbias).start()
        pltpu.make_async_copy(b1_hbm, b1_scr, sbias).wait()
        pltpu.make_async_copy(b2_hbm, b2_scr, sbias).wait()

    @pl.when((b < NB - 2))
    def _():
        for i in range(3):
            @pl.when((lax.rem(b + 2, 3) == i) & (meta_ref[b + 2, 7] == 1))
            def _(i=i):
                _xfetch(b + 2, i).start()

    @pl.when(b == 0)
    def _():
        _start(cur_e, 0)

    @pl.when((b == 0) & (has_nxt == 1))
    def _():
        _start(nxt_e, 1)

    for pp in range(3):
        @pl.when((ld == 1) & (has2 == 1) & (p == pp))
        def _(pp=pp):
            _start(nxt2_e, (pp + 2) % 3)

    for pp in range(3):
        @pl.when((ld == 1) & (p == pp))
        def _(pp=pp):
            _wait(cur_e, pp)

    p3 = lax.rem(b, 3)
    act = meta_ref[b, 7]
    for i in range(3):
        @pl.when((p3 == i) & (act == 1))
        def _(i=i):
            _xfetch(b, i).wait()

    @pl.when(meta_ref[b, 7] == 1)
    def _():
        for i in range(3):
            @pl.when(p3 == i)
            def _(i=i):
                x = x_scr[i]
                h = jnp.dot(x, w1_scr[p],
                            preferred_element_type=jnp.float32)
                h = h + b1_scr[cur_e]
                h = 0.5 * h * (1.0 + lax.erf(h * (1.0 / math.sqrt(2.0))))
                y = jnp.dot(h, w2_scr[p],
                            preferred_element_type=jnp.float32)
                y_scr[i] = y + b2_scr[cur_e]

    for i in range(3):
        @pl.when((p3 == i) & (act == 1))
        def _(i=i):
            _ywrite(b, i).start()

    @pl.when(b >= 2)
    def _():
        for i in range(3):
            @pl.when((lax.rem(b - 2, 3) == i) & (meta_ref[b - 2, 7] == 1))
            def _(i=i):
                _ywrite(b - 2, i).wait()

    @pl.when(b == NB - 1)
    def _():
        for j in (NB - 2, NB - 1):
            @pl.when(meta_ref[j, 7] == 1)
            def _(j=j):
                _ywrite(j, j % 3).wait()


def _ffn(meta, xs, W1, b1, W2, b2):
    grid_spec = pltpu.PrefetchScalarGridSpec(
        num_scalar_prefetch=1,
        grid=(NB,),
        in_specs=[
            pl.BlockSpec(memory_space=pl.ANY),
            pl.BlockSpec(memory_space=pl.ANY),
            pl.BlockSpec(memory_space=pl.ANY),
            pl.BlockSpec(memory_space=pl.ANY),
            pl.BlockSpec(memory_space=pl.ANY),
        ],
        out_specs=pl.BlockSpec(memory_space=pl.ANY),
        scratch_shapes=[
            pltpu.VMEM((3, D, D), jnp.float32),
            pltpu.VMEM((3, D, D), jnp.float32),
            pltpu.SemaphoreType.DMA,
            pltpu.SemaphoreType.DMA,
            pltpu.SemaphoreType.DMA,
            pltpu.VMEM((3, BT, D), jnp.float32),
            pltpu.VMEM((3, BT, D), jnp.float32),
            pltpu.VMEM((E, D), jnp.float32),
            pltpu.VMEM((E, D), jnp.float32),
            pltpu.SemaphoreType.DMA,
            pltpu.SemaphoreType.DMA,
            pltpu.SemaphoreType.DMA,
            pltpu.SemaphoreType.DMA,
            pltpu.SemaphoreType.DMA,
            pltpu.SemaphoreType.DMA,
            pltpu.SemaphoreType.DMA,
        ],
    )
    return pl.pallas_call(
        _ffn_body,
        grid_spec=grid_spec,
        out_shape=jax.ShapeDtypeStruct((NUM_SLOTS, D), jnp.float32),
    )(meta, xs, W1, b1, W2, b2)


_SC_CORES = 2
_SC_SUBCORES = 16


@functools.cache
def _make_sc_kernels():
    nw = _SC_CORES * _SC_SUBCORES
    bw = S // nw
    mesh = plsc.VectorSubcoreMesh(core_axis_name="c", subcore_axis_name="s")
    scratch = [
        pltpu.VMEM((bw,), jnp.int32),
        pltpu.VMEM((bw, D), jnp.float32),
        pltpu.SemaphoreType.DMA,
    ]

    @functools.partial(
        pl.kernel, mesh=mesh,
        out_type=jax.ShapeDtypeStruct((NUM_SLOTS, D), jnp.float32),
        scratch_types=scratch,
    )
    def dispatch(x_hbm, slot_hbm, xs_hbm, idx_v, rows_v, sem):
        wid = lax.axis_index("s") * _SC_CORES + lax.axis_index("c")
        base = wid * bw
        pltpu.sync_copy(slot_hbm.at[pl.ds(base, bw)], idx_v)
        pltpu.sync_copy(x_hbm.at[pl.ds(base, bw)], rows_v)
        pltpu.async_copy(rows_v, xs_hbm.at[idx_v], sem).wait()

    @functools.partial(
        pl.kernel, mesh=mesh,
        out_type=jax.ShapeDtypeStruct((S, D), jnp.float32),
        scratch_types=scratch,
    )
    def combine(ys_hbm, slot_hbm, out_hbm, idx_v, rows_v, sem):
        wid = lax.axis_index("s") * _SC_CORES + lax.axis_index("c")
        base = wid * bw
        pltpu.sync_copy(slot_hbm.at[pl.ds(base, bw)], idx_v)
        pltpu.async_copy(ys_hbm.at[idx_v], rows_v, sem).wait()
        pltpu.sync_copy(rows_v, out_hbm.at[pl.ds(base, bw)])

    return dispatch, combine


def kernel(x, Wg, bg, W1, b1, W2, b2):
    _dispatch, _combine = _make_sc_kernels()
    x2 = x.reshape(S, D)
    probs, slot, meta = _router(x2, Wg, bg.reshape(1, E))
    xs = _dispatch(x2, slot)
    ys = _ffn(meta, xs, W1, b1, W2, b2)
    out = _combine(ys, slot)
    return out.reshape(x.shape), probs.reshape(x.shape[0], S, E)
